# baseline (speedup 1.0000x reference)
import functools

import jax
import jax.numpy as jnp
from jax import lax
from jax.experimental import pallas as pl
from jax.experimental.pallas import tpu as pltpu
from jax.experimental.pallas import tpu_sc as plsc

B = 4096
NUM_IDS = 1000000
SEQ_LEN = 50
N_TOK = B * SEQ_LEN
ID_EMB = 32
ING_EMB = 32
ING_RAW = 64
OUT_SIZE = 64

CHUNK = 128
NW = 32
TOK_PER_W = N_TOK // NW
CPW = TOK_PER_W // CHUNK
CHUNKS_PER_L = B // CHUNK

TRB = 8192
S_STRIDE = 31 * TRB
N_TBLK = S_STRIDE // TRB
MAX_TBLK = (NUM_IDS + TRB - 1) // TRB - 1


def _pack_body(b0_ref, b1_ref, b2_ref, b3_ref, eye_ref, out_ref):
    i = pl.program_id(0)
    col = (3 * S_STRIDE + i * TRB
           + lax.broadcasted_iota(jnp.int32, (ID_EMB, TRB), 1))
    b3 = jnp.where(col < NUM_IDS, b3_ref[...], 0.0)
    x = jnp.concatenate(
        [b0_ref[...], b1_ref[...], b2_ref[...], b3], axis=0)
    out_ref[...] = lax.dot_general(
        x, eye_ref[...], (((0,), (0,)), ((), ())),
        preferred_element_type=jnp.float32)


def _tc_pack(tblT, eye):
    def mk(s):
        return pl.BlockSpec(
            (ID_EMB, TRB),
            lambda i: (0, jnp.minimum(s * N_TBLK + i, MAX_TBLK)))
    return pl.pallas_call(
        _pack_body,
        grid=(N_TBLK,),
        in_specs=[mk(0), mk(1), mk(2), mk(3),
                  pl.BlockSpec((128, 128), lambda i: (0, 0))],
        out_specs=pl.BlockSpec((TRB, 128), lambda i: (i, 0)),
        out_shape=jax.ShapeDtypeStruct((S_STRIDE, 128), jnp.float32),
    )(tblT, tblT, tblT, tblT, eye)


def _gather_body(table_hbm, idx_hbm, out_hbm, idx_v, rows_v, sem):
    nc = 2
    wid = lax.axis_index("s") * nc + lax.axis_index("c")

    pltpu.sync_copy(idx_hbm.at[wid], idx_v)

    def chunk_step(j, carry):
        pltpu.async_copy(table_hbm.at[idx_v.at[j]], rows_v, sem).wait()
        c = wid * CPW + j
        l = lax.shift_right_logical(c, 5)
        b0 = pl.multiple_of(
            lax.shift_left(lax.bitwise_and(c, CHUNKS_PER_L - 1), 7), CHUNK)
        pltpu.sync_copy(rows_v, out_hbm.at[l, pl.ds(b0, CHUNK),
                                           pl.ds(0, ID_EMB)])
        return carry

    lax.fori_loop(0, CPW, chunk_step, 0)


@jax.jit
def _sc_gather(table_lin, idx_3d):
    mesh = plsc.VectorSubcoreMesh(core_axis_name="c", subcore_axis_name="s")
    fn = pl.kernel(
        _gather_body,
        mesh=mesh,
        out_type=jax.ShapeDtypeStruct((SEQ_LEN, B, 128), jnp.float32),
        scratch_types=[
            pltpu.VMEM((CPW, CHUNK), jnp.int32),
            pltpu.VMEM((CHUNK, ID_EMB), jnp.float32),
            pltpu.SemaphoreType.DMA,
        ],
        compiler_params=pltpu.CompilerParams(use_tc_tiling_on_sc=False),
    )
    return fn(table_lin, idx_3d)


B2 = 1024


def _xing_body(ingt_ref, wingT_ref, bing_ref, out_ref):
    out_ref[0] = jnp.tanh(
        jnp.dot(wingT_ref[...], ingt_ref[0], preferred_element_type=jnp.float32)
        + bing_ref[...]
    )


def _tc_xing(ing_t, wingT, bing):
    grid = (SEQ_LEN, B // B2)
    return pl.pallas_call(
        _xing_body,
        grid=grid,
        in_specs=[
            pl.BlockSpec((1, ING_RAW, B2), lambda l, b: (l, 0, b)),
            pl.BlockSpec((ING_EMB, ING_RAW), lambda l, b: (0, 0)),
            pl.BlockSpec((ING_EMB, 1), lambda l, b: (0, 0)),
        ],
        out_specs=pl.BlockSpec((1, ING_EMB, B2), lambda l, b: (l, 0, b)),
        out_shape=jax.ShapeDtypeStruct((SEQ_LEN, ING_EMB, B), jnp.float32),
    )(ing_t, wingT, bing)


def _dense_body(xid_ref, xing_ref, w1T_ref, w2T_ref, bout_ref, out_ref):
    acc = lax.dot_general(
        w1T_ref[...], xid_ref[0][:, :ID_EMB], (((1,), (1,)), ((), ())),
        preferred_element_type=jnp.float32)
    acc = acc + jnp.dot(w2T_ref[...], xing_ref[0],
                        preferred_element_type=jnp.float32)
    out_ref[0] = jnp.tanh(acc + bout_ref[...])


def _tc_dense(x_id_lm, xing_t, w1T, w2T, bout):
    grid = (SEQ_LEN, B // B2)
    return pl.pallas_call(
        _dense_body,
        grid=grid,
        in_specs=[
            pl.BlockSpec((1, B2, 128), lambda l, b: (l, b, 0)),
            pl.BlockSpec((1, ING_EMB, B2), lambda l, b: (l, 0, b)),
            pl.BlockSpec((OUT_SIZE, ID_EMB), lambda l, b: (0, 0)),
            pl.BlockSpec((OUT_SIZE, ING_EMB), lambda l, b: (0, 0)),
            pl.BlockSpec((OUT_SIZE, 1), lambda l, b: (0, 0)),
        ],
        out_specs=pl.BlockSpec((1, OUT_SIZE, B2), lambda l, b: (l, 0, b)),
        out_shape=jax.ShapeDtypeStruct((SEQ_LEN, OUT_SIZE, B), jnp.float32),
    )(x_id_lm, xing_t, w1T, w2T, bout)


def kernel(recipe_id, ing, other_features, id_table, W_ing, b_ing, W_out, b_out):
    idx = recipe_id.astype(jnp.int32)
    s = ((idx >= S_STRIDE).astype(jnp.int32)
         + (idx >= 2 * S_STRIDE).astype(jnp.int32)
         + (idx >= 3 * S_STRIDE).astype(jnp.int32))
    p = ((idx - s * S_STRIDE) << 2) + s
    idx_3d = p.transpose(1, 0).reshape(NW, CPW, CHUNK)
    tblT = id_table.transpose(1, 0)
    packed = _tc_pack(tblT, jnp.eye(128, dtype=jnp.float32))
    table_lin = packed.reshape(4 * S_STRIDE, ID_EMB)
    x_id_lm = _sc_gather(table_lin, idx_3d)
    ing_t = jnp.transpose(ing, (1, 2, 0))
    xing_t = _tc_xing(ing_t, W_ing.T, b_ing.reshape(ING_EMB, 1))
    out_t = _tc_dense(
        x_id_lm, xing_t,
        W_out[:ID_EMB].T,
        W_out[ID_EMB:].T,
        b_out.reshape(OUT_SIZE, 1),
    )
    return jnp.transpose(out_t, (2, 0, 1))


# speedup vs baseline: 1.2654x; 1.2654x over previous
import functools

import jax
import jax.numpy as jnp
from jax import lax
from jax.experimental import pallas as pl
from jax.experimental.pallas import tpu as pltpu
from jax.experimental.pallas import tpu_sc as plsc

B = 4096
NUM_IDS = 1000000
SEQ_LEN = 50
N_TOK = B * SEQ_LEN
ID_EMB = 32
ING_EMB = 32
ING_RAW = 64
OUT_SIZE = 64

CHUNK = 128
NW = 32
TOK_PER_W = N_TOK // NW
CPW = TOK_PER_W // CHUNK
CHUNKS_PER_L = B // CHUNK

TRB = 8192
S_STRIDE = 31 * TRB
N_TBLK = S_STRIDE // TRB
MAX_TBLK = (NUM_IDS + TRB - 1) // TRB - 1


def _pack_body(b0_ref, b1_ref, b2_ref, b3_ref, eye_ref, out_ref):
    i = pl.program_id(0)
    col = (3 * S_STRIDE + i * TRB
           + lax.broadcasted_iota(jnp.int32, (ID_EMB, TRB), 1))
    b3 = jnp.where(col < NUM_IDS, b3_ref[...], 0.0)
    x = jnp.concatenate(
        [b0_ref[...], b1_ref[...], b2_ref[...], b3], axis=0)
    out_ref[...] = lax.dot_general(
        x, eye_ref[...], (((0,), (0,)), ((), ())),
        preferred_element_type=jnp.float32)


def _tc_pack(tblT, eye):
    def mk(s):
        return pl.BlockSpec(
            (ID_EMB, TRB),
            lambda i: (0, jnp.minimum(s * N_TBLK + i, MAX_TBLK)))
    return pl.pallas_call(
        _pack_body,
        grid=(N_TBLK,),
        in_specs=[mk(0), mk(1), mk(2), mk(3),
                  pl.BlockSpec((128, 128), lambda i: (0, 0))],
        out_specs=pl.BlockSpec((TRB, 128), lambda i: (i, 0)),
        out_shape=jax.ShapeDtypeStruct((S_STRIDE, 128), jnp.float32),
    )(tblT, tblT, tblT, tblT, eye)


def _gather_body(table_hbm, idx_hbm, out_hbm, idx_v, rows0, rows1, sem0, sem1):
    nc = 2
    wid = lax.axis_index("s") * nc + lax.axis_index("c")

    pltpu.sync_copy(idx_hbm.at[wid], idx_v)

    def start(j, rows, sem):
        pltpu.async_copy(table_hbm.at[idx_v.at[j]], rows, sem)

    def wait(j, rows, sem):
        pltpu.make_async_copy(table_hbm.at[idx_v.at[j]], rows, sem).wait()

    def write(j, rows):
        c = wid * CPW + j
        l = lax.shift_right_logical(c, 5)
        b0 = pl.multiple_of(
            lax.shift_left(lax.bitwise_and(c, CHUNKS_PER_L - 1), 7), CHUNK)
        pltpu.sync_copy(rows, out_hbm.at[l, pl.ds(b0, CHUNK),
                                         pl.ds(0, ID_EMB)])

    start(0, rows0, sem0)

    def body2(jj, carry):
        j0 = jj * 2
        start(j0 + 1, rows1, sem1)
        wait(j0, rows0, sem0)
        write(j0, rows0)

        @pl.when(jj + 1 < CPW // 2)
        def _():
            start(j0 + 2, rows0, sem0)

        wait(j0 + 1, rows1, sem1)
        write(j0 + 1, rows1)
        return carry

    lax.fori_loop(0, CPW // 2, body2, 0)


@jax.jit
def _sc_gather(table_lin, idx_3d):
    mesh = plsc.VectorSubcoreMesh(core_axis_name="c", subcore_axis_name="s")
    fn = pl.kernel(
        _gather_body,
        mesh=mesh,
        out_type=jax.ShapeDtypeStruct((SEQ_LEN, B, 128), jnp.float32),
        scratch_types=[
            pltpu.VMEM((CPW, CHUNK), jnp.int32),
            pltpu.VMEM((CHUNK, ID_EMB), jnp.float32),
            pltpu.VMEM((CHUNK, ID_EMB), jnp.float32),
            pltpu.SemaphoreType.DMA,
            pltpu.SemaphoreType.DMA,
        ],
        compiler_params=pltpu.CompilerParams(use_tc_tiling_on_sc=False),
    )
    return fn(table_lin, idx_3d)


B2 = 1024


def _dense_body(xid_ref, ingt_ref, wingT_ref, bing_ref, w1T_ref, w2T_ref,
                bout_ref, out_ref):
    xing = jnp.tanh(
        jnp.dot(wingT_ref[...], ingt_ref[0], preferred_element_type=jnp.float32)
        + bing_ref[...]
    )
    acc = lax.dot_general(
        w1T_ref[...], xid_ref[0][:, :ID_EMB], (((1,), (1,)), ((), ())),
        preferred_element_type=jnp.float32)
    acc = acc + jnp.dot(w2T_ref[...], xing, preferred_element_type=jnp.float32)
    out_ref[0] = jnp.tanh(acc + bout_ref[...])


def _tc_dense(x_id_lm, ing_t, wingT, bing, w1T, w2T, bout):
    grid = (SEQ_LEN, B // B2)
    return pl.pallas_call(
        _dense_body,
        grid=grid,
        in_specs=[
            pl.BlockSpec((1, B2, 128), lambda l, b: (l, b, 0)),
            pl.BlockSpec((1, ING_RAW, B2), lambda l, b: (l, 0, b)),
            pl.BlockSpec((ING_EMB, ING_RAW), lambda l, b: (0, 0)),
            pl.BlockSpec((ING_EMB, 1), lambda l, b: (0, 0)),
            pl.BlockSpec((OUT_SIZE, ID_EMB), lambda l, b: (0, 0)),
            pl.BlockSpec((OUT_SIZE, ING_EMB), lambda l, b: (0, 0)),
            pl.BlockSpec((OUT_SIZE, 1), lambda l, b: (0, 0)),
        ],
        out_specs=pl.BlockSpec((1, OUT_SIZE, B2), lambda l, b: (l, 0, b)),
        out_shape=jax.ShapeDtypeStruct((SEQ_LEN, OUT_SIZE, B), jnp.float32),
    )(x_id_lm, ing_t, wingT, bing, w1T, w2T, bout)


def kernel(recipe_id, ing, other_features, id_table, W_ing, b_ing, W_out, b_out):
    idx = recipe_id.astype(jnp.int32)
    s = ((idx >= S_STRIDE).astype(jnp.int32)
         + (idx >= 2 * S_STRIDE).astype(jnp.int32)
         + (idx >= 3 * S_STRIDE).astype(jnp.int32))
    p = ((idx - s * S_STRIDE) << 2) + s
    idx_3d = p.transpose(1, 0).reshape(NW, CPW, CHUNK)
    tblT = id_table.transpose(1, 0)
    packed = _tc_pack(tblT, jnp.eye(128, dtype=jnp.float32))
    table_lin = packed.reshape(4 * S_STRIDE, ID_EMB)
    x_id_lm = _sc_gather(table_lin, idx_3d)
    ing_t = jnp.transpose(ing, (1, 2, 0))
    out_t = _tc_dense(
        x_id_lm, ing_t,
        W_ing.T,
        b_ing.reshape(ING_EMB, 1),
        W_out[:ID_EMB].T,
        W_out[ID_EMB:].T,
        b_out.reshape(OUT_SIZE, 1),
    )
    return jnp.transpose(out_t, (2, 0, 1))


# speedup vs baseline: 1.5558x; 1.2295x over previous
import functools

import jax
import jax.numpy as jnp
from jax import lax
from jax.experimental import pallas as pl
from jax.experimental.pallas import tpu as pltpu
from jax.experimental.pallas import tpu_sc as plsc

B = 4096
NUM_IDS = 1000000
SEQ_LEN = 50
N_TOK = B * SEQ_LEN
ID_EMB = 32
ING_EMB = 32
ING_RAW = 64
OUT_SIZE = 64

CHUNK = 128
NW = 32
TOK_PER_W = N_TOK // NW
CPW = TOK_PER_W // CHUNK
CHUNKS_PER_L = B // CHUNK

TRB = 8192
S_STRIDE = 31 * TRB
N_TBLK = S_STRIDE // TRB
MAX_TBLK = (NUM_IDS + TRB - 1) // TRB - 1


def _pack_body(b0_ref, b1_ref, b2_ref, b3_ref, eye_ref, out_ref):
    i = pl.program_id(0)
    col = (3 * S_STRIDE + i * TRB
           + lax.broadcasted_iota(jnp.int32, (ID_EMB, TRB), 1))
    b3 = jnp.where(col < NUM_IDS, b3_ref[...], 0.0)
    x = jnp.concatenate(
        [b0_ref[...], b1_ref[...], b2_ref[...], b3], axis=0)
    out_ref[...] = lax.dot_general(
        x, eye_ref[...], (((0,), (0,)), ((), ())),
        preferred_element_type=jnp.float32)


def _tc_pack(tblT, eye):
    def mk(s):
        return pl.BlockSpec(
            (ID_EMB, TRB),
            lambda i: (0, jnp.minimum(s * N_TBLK + i, MAX_TBLK)))
    return pl.pallas_call(
        _pack_body,
        grid=(N_TBLK,),
        in_specs=[mk(0), mk(1), mk(2), mk(3),
                  pl.BlockSpec((128, 128), lambda i: (0, 0))],
        out_specs=pl.BlockSpec((TRB, 128), lambda i: (i, 0)),
        out_shape=jax.ShapeDtypeStruct((S_STRIDE, 128), jnp.float32),
    )(tblT, tblT, tblT, tblT, eye)


def _gather_body(table_hbm, idx_hbm, out_hbm, idx_v, rows0, rows1, sem0, sem1):
    nc = 2
    wid = lax.axis_index("s") * nc + lax.axis_index("c")

    pltpu.sync_copy(idx_hbm.at[wid], idx_v)

    def start(j, rows, sem):
        pltpu.async_copy(table_hbm.at[idx_v.at[j]], rows, sem)

    def wait(j, rows, sem):
        pltpu.make_async_copy(table_hbm.at[idx_v.at[j]], rows, sem).wait()

    def write(j, rows):
        c = wid * CPW + j
        l = lax.shift_right_logical(c, 5)
        b0 = pl.multiple_of(
            lax.shift_left(lax.bitwise_and(c, CHUNKS_PER_L - 1), 7), CHUNK)
        pltpu.sync_copy(rows, out_hbm.at[l, pl.ds(b0, CHUNK),
                                         pl.ds(0, ID_EMB)])

    start(0, rows0, sem0)

    def body2(jj, carry):
        j0 = jj * 2
        start(j0 + 1, rows1, sem1)
        wait(j0, rows0, sem0)
        write(j0, rows0)

        @pl.when(jj + 1 < CPW // 2)
        def _():
            start(j0 + 2, rows0, sem0)

        wait(j0 + 1, rows1, sem1)
        write(j0 + 1, rows1)
        return carry

    lax.fori_loop(0, CPW // 2, body2, 0)


@jax.jit
def _sc_gather(table_lin, idx_3d):
    mesh = plsc.VectorSubcoreMesh(core_axis_name="c", subcore_axis_name="s")
    fn = pl.kernel(
        _gather_body,
        mesh=mesh,
        out_type=jax.ShapeDtypeStruct((SEQ_LEN, B, 128), jnp.float32),
        scratch_types=[
            pltpu.VMEM((CPW, CHUNK), jnp.int32),
            pltpu.VMEM((CHUNK, ID_EMB), jnp.float32),
            pltpu.VMEM((CHUNK, ID_EMB), jnp.float32),
            pltpu.SemaphoreType.DMA,
            pltpu.SemaphoreType.DMA,
        ],
        compiler_params=pltpu.CompilerParams(use_tc_tiling_on_sc=False),
    )
    return fn(table_lin, idx_3d)


B2 = 2048


def _dense_body(xid_ref, ingt_ref, wingT_ref, bing_ref, w1T_ref, w2T_ref,
                bout_ref, out_ref):
    xing = jnp.tanh(
        jnp.dot(wingT_ref[...], ingt_ref[0], preferred_element_type=jnp.float32)
        + bing_ref[...]
    )
    acc = lax.dot_general(
        w1T_ref[...], xid_ref[0][:, :ID_EMB], (((1,), (1,)), ((), ())),
        preferred_element_type=jnp.float32)
    acc = acc + jnp.dot(w2T_ref[...], xing, preferred_element_type=jnp.float32)
    out_ref[0] = jnp.tanh(acc + bout_ref[...])


def _tc_dense(x_id_lm, ing_t, wingT, bing, w1T, w2T, bout):
    grid = (SEQ_LEN, B // B2)
    return pl.pallas_call(
        _dense_body,
        grid=grid,
        in_specs=[
            pl.BlockSpec((1, B2, 128), lambda l, b: (l, b, 0)),
            pl.BlockSpec((1, ING_RAW, B2), lambda l, b: (l, 0, b)),
            pl.BlockSpec((ING_EMB, ING_RAW), lambda l, b: (0, 0)),
            pl.BlockSpec((ING_EMB, 1), lambda l, b: (0, 0)),
            pl.BlockSpec((OUT_SIZE, ID_EMB), lambda l, b: (0, 0)),
            pl.BlockSpec((OUT_SIZE, ING_EMB), lambda l, b: (0, 0)),
            pl.BlockSpec((OUT_SIZE, 1), lambda l, b: (0, 0)),
        ],
        out_specs=pl.BlockSpec((1, OUT_SIZE, B2), lambda l, b: (l, 0, b)),
        out_shape=jax.ShapeDtypeStruct((SEQ_LEN, OUT_SIZE, B), jnp.float32),
    )(x_id_lm, ing_t, wingT, bing, w1T, w2T, bout)


def kernel(recipe_id, ing, other_features, id_table, W_ing, b_ing, W_out, b_out):
    idx = recipe_id.astype(jnp.int32)
    s = ((idx >= S_STRIDE).astype(jnp.int32)
         + (idx >= 2 * S_STRIDE).astype(jnp.int32)
         + (idx >= 3 * S_STRIDE).astype(jnp.int32))
    p = ((idx - s * S_STRIDE) << 2) + s
    idx_3d = p.transpose(1, 0).reshape(NW, CPW, CHUNK)
    tblT = id_table.transpose(1, 0)
    packed = _tc_pack(tblT, jnp.eye(128, dtype=jnp.float32))
    table_lin = packed.reshape(4 * S_STRIDE, ID_EMB)
    x_id_lm = _sc_gather(table_lin, idx_3d)
    ing_t = jnp.transpose(ing, (1, 2, 0))
    out_t = _tc_dense(
        x_id_lm, ing_t,
        W_ing.T,
        b_ing.reshape(ING_EMB, 1),
        W_out[:ID_EMB].T,
        W_out[ID_EMB:].T,
        b_out.reshape(OUT_SIZE, 1),
    )
    return jnp.transpose(out_t, (2, 0, 1))


# speedup vs baseline: 1.8012x; 1.1578x over previous
import functools

import jax
import jax.numpy as jnp
from jax import lax
from jax.experimental import pallas as pl
from jax.experimental.pallas import tpu as pltpu
from jax.experimental.pallas import tpu_sc as plsc

B = 4096
NUM_IDS = 1000000
SEQ_LEN = 50
N_TOK = B * SEQ_LEN
ID_EMB = 32
ING_EMB = 32
ING_RAW = 64
OUT_SIZE = 64

CHUNK = 128
NW = 32
TOK_PER_W = N_TOK // NW
CPW = TOK_PER_W // CHUNK
CHUNKS_PER_L = B // CHUNK

TRB = 8192
S_STRIDE = 31 * TRB
N_TBLK = S_STRIDE // TRB
MAX_TBLK = (NUM_IDS + TRB - 1) // TRB - 1


def _pack_body(b0_ref, b1_ref, b2_ref, b3_ref, eye_ref, out_ref):
    i = pl.program_id(0)
    col = (3 * S_STRIDE + i * TRB
           + lax.broadcasted_iota(jnp.int32, (ID_EMB, TRB), 1))
    b3 = jnp.where(col < NUM_IDS, b3_ref[...], 0.0)
    x = jnp.concatenate(
        [b0_ref[...], b1_ref[...], b2_ref[...], b3], axis=0)
    out_ref[...] = lax.dot_general(
        x, eye_ref[...], (((0,), (0,)), ((), ())),
        preferred_element_type=jnp.float32)


def _tc_pack(tblT, eye):
    def mk(s):
        return pl.BlockSpec(
            (ID_EMB, TRB),
            lambda i: (0, jnp.minimum(s * N_TBLK + i, MAX_TBLK)))
    return pl.pallas_call(
        _pack_body,
        grid=(N_TBLK,),
        in_specs=[mk(0), mk(1), mk(2), mk(3),
                  pl.BlockSpec((128, 128), lambda i: (0, 0))],
        out_specs=pl.BlockSpec((TRB, 128), lambda i: (i, 0)),
        out_shape=jax.ShapeDtypeStruct((S_STRIDE, 128), jnp.float32),
    )(tblT, tblT, tblT, tblT, eye)


def _gather_body(table_hbm, idx_hbm, out_hbm, idx_v, rows0, rows1, sem0, sem1):
    nc = 2
    wid = lax.axis_index("s") * nc + lax.axis_index("c")

    pltpu.sync_copy(idx_hbm.at[wid], idx_v)

    def start(j, rows, sem):
        pltpu.async_copy(table_hbm.at[idx_v.at[j]], rows, sem)

    def wait(j, rows, sem):
        pltpu.make_async_copy(table_hbm.at[idx_v.at[j]], rows, sem).wait()

    def write(j, rows):
        c = wid * CPW + j
        l = lax.shift_right_logical(c, 5)
        b0 = pl.multiple_of(
            lax.shift_left(lax.bitwise_and(c, CHUNKS_PER_L - 1), 7), CHUNK)
        pltpu.sync_copy(rows, out_hbm.at[l, pl.ds(b0, CHUNK),
                                         pl.ds(0, ID_EMB)])

    start(0, rows0, sem0)

    def body2(jj, carry):
        j0 = jj * 2
        start(j0 + 1, rows1, sem1)
        wait(j0, rows0, sem0)
        write(j0, rows0)

        @pl.when(jj + 1 < CPW // 2)
        def _():
            start(j0 + 2, rows0, sem0)

        wait(j0 + 1, rows1, sem1)
        write(j0 + 1, rows1)
        return carry

    lax.fori_loop(0, CPW // 2, body2, 0)


@jax.jit
def _sc_gather(table_lin, idx_3d):
    mesh = plsc.VectorSubcoreMesh(core_axis_name="c", subcore_axis_name="s")
    fn = pl.kernel(
        _gather_body,
        mesh=mesh,
        out_type=jax.ShapeDtypeStruct((SEQ_LEN, B, 128), jnp.float32),
        scratch_types=[
            pltpu.VMEM((CPW, CHUNK), jnp.int32),
            pltpu.VMEM((CHUNK, ID_EMB), jnp.float32),
            pltpu.VMEM((CHUNK, ID_EMB), jnp.float32),
            pltpu.SemaphoreType.DMA,
            pltpu.SemaphoreType.DMA,
        ],
        compiler_params=pltpu.CompilerParams(use_tc_tiling_on_sc=False),
    )
    return fn(table_lin, idx_3d)


B2 = 4096


def _dense_body(xid_ref, ingt_ref, wingT_ref, bing_ref, w1T_ref, w2T_ref,
                bout_ref, out_ref):
    xing = jnp.tanh(
        jnp.dot(wingT_ref[...], ingt_ref[0], preferred_element_type=jnp.float32)
        + bing_ref[...]
    )
    acc = lax.dot_general(
        w1T_ref[...], xid_ref[0][:, :ID_EMB], (((1,), (1,)), ((), ())),
        preferred_element_type=jnp.float32)
    acc = acc + jnp.dot(w2T_ref[...], xing, preferred_element_type=jnp.float32)
    out_ref[0] = jnp.tanh(acc + bout_ref[...])


def _tc_dense(x_id_lm, ing_t, wingT, bing, w1T, w2T, bout):
    grid = (SEQ_LEN, B // B2)
    return pl.pallas_call(
        _dense_body,
        grid=grid,
        in_specs=[
            pl.BlockSpec((1, B2, 128), lambda l, b: (l, b, 0)),
            pl.BlockSpec((1, ING_RAW, B2), lambda l, b: (l, 0, b)),
            pl.BlockSpec((ING_EMB, ING_RAW), lambda l, b: (0, 0)),
            pl.BlockSpec((ING_EMB, 1), lambda l, b: (0, 0)),
            pl.BlockSpec((OUT_SIZE, ID_EMB), lambda l, b: (0, 0)),
            pl.BlockSpec((OUT_SIZE, ING_EMB), lambda l, b: (0, 0)),
            pl.BlockSpec((OUT_SIZE, 1), lambda l, b: (0, 0)),
        ],
        out_specs=pl.BlockSpec((1, OUT_SIZE, B2), lambda l, b: (l, 0, b)),
        out_shape=jax.ShapeDtypeStruct((SEQ_LEN, OUT_SIZE, B), jnp.float32),
    )(x_id_lm, ing_t, wingT, bing, w1T, w2T, bout)


def kernel(recipe_id, ing, other_features, id_table, W_ing, b_ing, W_out, b_out):
    idx = recipe_id.astype(jnp.int32)
    s = ((idx >= S_STRIDE).astype(jnp.int32)
         + (idx >= 2 * S_STRIDE).astype(jnp.int32)
         + (idx >= 3 * S_STRIDE).astype(jnp.int32))
    p = ((idx - s * S_STRIDE) << 2) + s
    idx_3d = p.transpose(1, 0).reshape(NW, CPW, CHUNK)
    tblT = id_table.transpose(1, 0)
    packed = _tc_pack(tblT, jnp.eye(128, dtype=jnp.float32))
    table_lin = packed.reshape(4 * S_STRIDE, ID_EMB)
    x_id_lm = _sc_gather(table_lin, idx_3d)
    ing_t = jnp.transpose(ing, (1, 2, 0))
    out_t = _tc_dense(
        x_id_lm, ing_t,
        W_ing.T,
        b_ing.reshape(ING_EMB, 1),
        W_out[:ID_EMB].T,
        W_out[ID_EMB:].T,
        b_out.reshape(OUT_SIZE, 1),
    )
    return jnp.transpose(out_t, (2, 0, 1))


# speedup vs baseline: 1.8125x; 1.0062x over previous
import functools

import jax
import jax.numpy as jnp
from jax import lax
from jax.experimental import pallas as pl
from jax.experimental.pallas import tpu as pltpu
from jax.experimental.pallas import tpu_sc as plsc

B = 4096
NUM_IDS = 1000000
SEQ_LEN = 50
N_TOK = B * SEQ_LEN
ID_EMB = 32
ING_EMB = 32
ING_RAW = 64
OUT_SIZE = 64

CHUNK = 128
NW = 32
TOK_PER_W = N_TOK // NW
CPW = TOK_PER_W // CHUNK
CHUNKS_PER_L = B // CHUNK

TRB = 16384
S_STRIDE = 16 * TRB
N_TBLK = S_STRIDE // TRB
MAX_TBLK = (NUM_IDS + TRB - 1) // TRB - 1


def _pack_body(b0_ref, b1_ref, b2_ref, b3_ref, eye_ref, out_ref):
    i = pl.program_id(0)
    col = (3 * S_STRIDE + i * TRB
           + lax.broadcasted_iota(jnp.int32, (ID_EMB, TRB), 1))
    b3 = jnp.where(col < NUM_IDS, b3_ref[...], 0.0)
    x = jnp.concatenate(
        [b0_ref[...], b1_ref[...], b2_ref[...], b3], axis=0)
    out_ref[...] = lax.dot_general(
        x, eye_ref[...], (((0,), (0,)), ((), ())),
        preferred_element_type=jnp.float32)


def _tc_pack(tblT, eye):
    def mk(s):
        return pl.BlockSpec(
            (ID_EMB, TRB),
            lambda i: (0, jnp.minimum(s * N_TBLK + i, MAX_TBLK)))
    return pl.pallas_call(
        _pack_body,
        grid=(N_TBLK,),
        in_specs=[mk(0), mk(1), mk(2), mk(3),
                  pl.BlockSpec((128, 128), lambda i: (0, 0))],
        out_specs=pl.BlockSpec((TRB, 128), lambda i: (i, 0)),
        out_shape=jax.ShapeDtypeStruct((S_STRIDE, 128), jnp.float32),
    )(tblT, tblT, tblT, tblT, eye)


def _gather_body(table_hbm, idx_hbm, out_hbm, idx_v, rows0, rows1, sem0, sem1):
    nc = 2
    wid = lax.axis_index("s") * nc + lax.axis_index("c")

    pltpu.sync_copy(idx_hbm.at[wid], idx_v)

    def start(j, rows, sem):
        pltpu.async_copy(table_hbm.at[idx_v.at[j]], rows, sem)

    def wait(j, rows, sem):
        pltpu.make_async_copy(table_hbm.at[idx_v.at[j]], rows, sem).wait()

    def write(j, rows):
        c = wid * CPW + j
        l = lax.shift_right_logical(c, 5)
        b0 = pl.multiple_of(
            lax.shift_left(lax.bitwise_and(c, CHUNKS_PER_L - 1), 7), CHUNK)
        pltpu.sync_copy(rows, out_hbm.at[l, pl.ds(b0, CHUNK),
                                         pl.ds(0, ID_EMB)])

    start(0, rows0, sem0)

    def body2(jj, carry):
        j0 = jj * 2
        start(j0 + 1, rows1, sem1)
        wait(j0, rows0, sem0)
        write(j0, rows0)

        @pl.when(jj + 1 < CPW // 2)
        def _():
            start(j0 + 2, rows0, sem0)

        wait(j0 + 1, rows1, sem1)
        write(j0 + 1, rows1)
        return carry

    lax.fori_loop(0, CPW // 2, body2, 0)


@jax.jit
def _sc_gather(table_lin, idx_3d):
    mesh = plsc.VectorSubcoreMesh(core_axis_name="c", subcore_axis_name="s")
    fn = pl.kernel(
        _gather_body,
        mesh=mesh,
        out_type=jax.ShapeDtypeStruct((SEQ_LEN, B, 128), jnp.float32),
        scratch_types=[
            pltpu.VMEM((CPW, CHUNK), jnp.int32),
            pltpu.VMEM((CHUNK, ID_EMB), jnp.float32),
            pltpu.VMEM((CHUNK, ID_EMB), jnp.float32),
            pltpu.SemaphoreType.DMA,
            pltpu.SemaphoreType.DMA,
        ],
        compiler_params=pltpu.CompilerParams(use_tc_tiling_on_sc=False),
    )
    return fn(table_lin, idx_3d)


B2 = 4096


def _dense_body(xid_ref, ingt_ref, wingT_ref, bing_ref, w1T_ref, w2T_ref,
                bout_ref, out_ref):
    xing = jnp.tanh(
        jnp.dot(wingT_ref[...], ingt_ref[0], preferred_element_type=jnp.float32)
        + bing_ref[...]
    )
    acc = lax.dot_general(
        w1T_ref[...], xid_ref[0][:, :ID_EMB], (((1,), (1,)), ((), ())),
        preferred_element_type=jnp.float32)
    acc = acc + jnp.dot(w2T_ref[...], xing, preferred_element_type=jnp.float32)
    out_ref[0] = jnp.tanh(acc + bout_ref[...])


def _tc_dense(x_id_lm, ing_t, wingT, bing, w1T, w2T, bout):
    grid = (SEQ_LEN, B // B2)
    return pl.pallas_call(
        _dense_body,
        grid=grid,
        in_specs=[
            pl.BlockSpec((1, B2, 128), lambda l, b: (l, b, 0)),
            pl.BlockSpec((1, ING_RAW, B2), lambda l, b: (l, 0, b)),
            pl.BlockSpec((ING_EMB, ING_RAW), lambda l, b: (0, 0)),
            pl.BlockSpec((ING_EMB, 1), lambda l, b: (0, 0)),
            pl.BlockSpec((OUT_SIZE, ID_EMB), lambda l, b: (0, 0)),
            pl.BlockSpec((OUT_SIZE, ING_EMB), lambda l, b: (0, 0)),
            pl.BlockSpec((OUT_SIZE, 1), lambda l, b: (0, 0)),
        ],
        out_specs=pl.BlockSpec((1, OUT_SIZE, B2), lambda l, b: (l, 0, b)),
        out_shape=jax.ShapeDtypeStruct((SEQ_LEN, OUT_SIZE, B), jnp.float32),
    )(x_id_lm, ing_t, wingT, bing, w1T, w2T, bout)


def kernel(recipe_id, ing, other_features, id_table, W_ing, b_ing, W_out, b_out):
    idx = recipe_id.astype(jnp.int32)
    s = ((idx >= S_STRIDE).astype(jnp.int32)
         + (idx >= 2 * S_STRIDE).astype(jnp.int32)
         + (idx >= 3 * S_STRIDE).astype(jnp.int32))
    p = ((idx - s * S_STRIDE) << 2) + s
    idx_3d = p.transpose(1, 0).reshape(NW, CPW, CHUNK)
    tblT = id_table.transpose(1, 0)
    packed = _tc_pack(tblT, jnp.eye(128, dtype=jnp.float32))
    table_lin = packed.reshape(4 * S_STRIDE, ID_EMB)
    x_id_lm = _sc_gather(table_lin, idx_3d)
    ing_t = jnp.transpose(ing, (1, 2, 0))
    out_t = _tc_dense(
        x_id_lm, ing_t,
        W_ing.T,
        b_ing.reshape(ING_EMB, 1),
        W_out[:ID_EMB].T,
        W_out[ID_EMB:].T,
        b_out.reshape(OUT_SIZE, 1),
    )
    return jnp.transpose(out_t, (2, 0, 1))
